# baseline (speedup 1.0000x reference)
import functools
import jax
import jax.numpy as jnp
from jax import lax
from jax.experimental import pallas as pl
from jax.experimental.pallas import tpu as pltpu
from jax.experimental.pallas import tpu_sc as plsc

N_USER = 25000
N_ITEM = 25000
N = N_USER + N_ITEM
D = 64
H = D // 2
NS = 16
L = 16
C = 128
SUP = 8
NG = 8
GLEAD = 6
NM = 2
N_PAD = 50176
R_PER_TILE = N_PAD // NS
RC = 112
PK = plsc.PackFormat.INTERLEAVED


def _make_sc_body(n_supers):
  def _sc_body(tab2, src2, dst2, val2,
               final, tabbf, buf0, buf1,
               acc, src_sv, dst_sv, val_sv,
               b0_v, b1_v, b2_v, b3_v, b4_v, b5_v, b6_v, b7_v,
               m0_v, m1_v,
               g0_s, g1_s, g2_s, g3_s, g4_s, g5_s, g6_s, g7_s,
               s0_s, s1_s, i_s):
    cid = lax.axis_index("c")
    sid = lax.axis_index("s")
    bfs = [b0_v, b1_v, b2_v, b3_v, b4_v, b5_v, b6_v, b7_v]
    msg = [m0_v, m1_v]
    gsem = [g0_s, g1_s, g2_s, g3_s, g4_s, g5_s, g6_s, g7_s]
    ssem = [s0_s, s1_s]
    zeros16 = jnp.zeros((L,), jnp.float32)
    row_off = cid * N_PAD
    rbase = sid * R_PER_TILE
    erow_base = sid * (n_supers * SUP)
    n_rc = R_PER_TILE // RC

    def fill_zero(buf):
        def zbody(r, carry):
            buf[r, pl.ds(0, L)] = zeros16
            buf[r, pl.ds(L, L)] = zeros16
            return carry
        lax.fori_loop(0, RC, zbody, 0)

    fill_zero(m0_v)

    def clear_acc(i, carry):
        pltpu.sync_copy(m0_v.at[pl.ds(0, RC)],
                        acc.at[pl.ds(rbase + i * RC, RC)])
        return carry
    lax.fori_loop(0, n_rc, clear_acc, 0)

    def pack_rows(fbuf, bbuf):
        def pbody(r, carry):
            p = plsc.pack(fbuf[r, pl.ds(0, L)], fbuf[r, pl.ds(L, L)],
                          format=PK)
            bbuf[r, pl.ds(0, 2 * L)] = p
            return carry
        lax.fori_loop(0, RC, pbody, 0)

    def pack_tab(i, carry):
        g0 = row_off + rbase + i * RC
        pltpu.sync_copy(tab2.at[pl.ds(g0, RC)], m1_v.at[pl.ds(0, RC)])
        pack_rows(m1_v, b0_v)
        pltpu.sync_copy(b0_v.at[pl.ds(0, RC)], tabbf.at[pl.ds(g0, RC)])
        return carry
    lax.fori_loop(0, n_rc, pack_tab, 0)
    plsc.subcore_barrier()

    def gissue(k, r, src_tab):
        pltpu.async_copy(src_tab.at[src_sv.at[r]], bfs[k], gsem[k])

    def gwait(k, src_tab):
        pltpu.make_async_copy(src_tab.at[src_sv.at[0]], bfs[k],
                              gsem[k]).wait()

    def sissue(m, r):
        pltpu.async_copy(msg[m], acc.at[dst_sv.at[r]], ssem[m], add=True)

    def swait(m):
        pltpu.make_async_copy(msg[m], acc.at[dst_sv.at[0]], ssem[m]).wait()

    def scale(k, m, r):
        src = bfs[k]
        dst = msg[m]

        def gb(g, carry):
            vseg = val_sv[r, pl.ds(g * L, L)]
            for kk in range(L):
                v = vseg[kk]
                e = g * L + kk
                a, b = plsc.unpack(src[e, pl.ds(0, 2 * L)], format=PK)
                dst[e, pl.ds(0, L)] = a * v
                dst[e, pl.ds(L, L)] = b * v
            return carry
        lax.fori_loop(0, C // L, gb, 0)

    def do_edges(src_tab):
        def super_body(s, carry):
            erow0 = erow_base + s * SUP
            pltpu.async_copy(src2.at[pl.ds(erow0, SUP)], src_sv, i_s)
            pltpu.async_copy(dst2.at[pl.ds(erow0, SUP)], dst_sv, i_s)
            pltpu.async_copy(val2.at[pl.ds(erow0, SUP)], val_sv, i_s)
            pltpu.make_async_copy(src2.at[pl.ds(erow0, SUP)], src_sv,
                                  i_s).wait()
            pltpu.make_async_copy(dst2.at[pl.ds(erow0, SUP)], dst_sv,
                                  i_s).wait()
            pltpu.make_async_copy(val2.at[pl.ds(erow0, SUP)], val_sv,
                                  i_s).wait()

            def adj_body(r, c2):
                for j in range(C // L):
                    src_sv[r, pl.ds(j * L, L)] = (
                        src_sv[r, pl.ds(j * L, L)] + row_off)
                return c2
            lax.fori_loop(0, SUP, adj_body, 0)

            for k in range(GLEAD):
                gissue(k, k, src_tab)

            for t in range(SUP):
                k = t % NG
                m = t % NM
                ta = t + GLEAD
                if ta < SUP:
                    gissue(ta % NG, ta, src_tab)
                gwait(k, src_tab)
                if t >= NM:
                    swait(m)
                scale(k, m, t)
                sissue(m, t)
            for t in range(SUP - NM, SUP):
                swait(t % NM)
            return carry
        lax.fori_loop(0, n_supers, super_body, 0)
        plsc.subcore_barrier()

    def export_layer(dst_buf):
        fill_zero(m1_v)

        def eb(i, carry):
            r0 = rbase + i * RC
            pltpu.sync_copy(acc.at[pl.ds(r0, RC)], m0_v.at[pl.ds(0, RC)])
            pack_rows(m0_v, b0_v)
            pltpu.sync_copy(b0_v.at[pl.ds(0, RC)],
                            dst_buf.at[pl.ds(row_off + r0, RC)])
            pltpu.sync_copy(m1_v.at[pl.ds(0, RC)], acc.at[pl.ds(r0, RC)])
            return carry
        lax.fori_loop(0, n_rc, eb, 0)
        plsc.subcore_barrier()

    do_edges(tabbf)
    export_layer(buf0)
    do_edges(buf0)
    export_layer(buf1)
    do_edges(buf1)

    quarter = jnp.float32(0.25)

    def mean_body(i, carry):
        r0 = rbase + i * RC
        g0 = row_off + r0
        pltpu.sync_copy(acc.at[pl.ds(r0, RC)], m0_v.at[pl.ds(0, RC)])
        pltpu.sync_copy(tab2.at[pl.ds(g0, RC)], m1_v.at[pl.ds(0, RC)])
        pltpu.sync_copy(buf0.at[pl.ds(g0, RC)], b0_v.at[pl.ds(0, RC)])
        pltpu.sync_copy(buf1.at[pl.ds(g0, RC)], b1_v.at[pl.ds(0, RC)])

        def rbody(r, rcarry):
            a0, a1 = plsc.unpack(b0_v[r, pl.ds(0, 2 * L)], format=PK)
            c0, c1 = plsc.unpack(b1_v[r, pl.ds(0, 2 * L)], format=PK)
            s0 = (m0_v[r, pl.ds(0, L)] + m1_v[r, pl.ds(0, L)] + a0 + c0)
            s1 = (m0_v[r, pl.ds(L, L)] + m1_v[r, pl.ds(L, L)] + a1 + c1)
            m0_v[r, pl.ds(0, L)] = s0 * quarter
            m0_v[r, pl.ds(L, L)] = s1 * quarter
            return rcarry
        lax.fori_loop(0, RC, rbody, 0)
        pltpu.sync_copy(m0_v.at[pl.ds(0, RC)], final.at[pl.ds(g0, RC)])
        return carry
    lax.fori_loop(0, n_rc, mean_body, 0)

  return _sc_body


@functools.partial(jax.jit, static_argnames=("n_supers",))
def _run(tab2, src2, dst2, val2, n_supers):
    mesh = plsc.VectorSubcoreMesh(core_axis_name="c", subcore_axis_name="s")
    f32 = jnp.float32
    i32 = jnp.int32
    bf16 = jnp.bfloat16
    out_type = (
        jax.ShapeDtypeStruct((2 * N_PAD, H), f32),
        jax.ShapeDtypeStruct((2 * N_PAD, H), bf16),
        jax.ShapeDtypeStruct((2 * N_PAD, H), bf16),
        jax.ShapeDtypeStruct((2 * N_PAD, H), bf16),
    )
    scratch = (
        [pltpu.VMEM_SHARED((N_PAD, H), f32)]
        + [pltpu.VMEM((SUP, C), i32),
           pltpu.VMEM((SUP, C), i32),
           pltpu.VMEM((SUP, C), f32)]
        + [pltpu.VMEM((C, H), bf16)] * NG
        + [pltpu.VMEM((C, H), f32)] * NM
        + [pltpu.SemaphoreType.DMA] * (NG + NM + 1)
    )
    run = pl.kernel(
        _make_sc_body(n_supers),
        out_type=out_type,
        mesh=mesh,
        scratch_types=scratch,
        compiler_params=pltpu.CompilerParams(use_tc_tiling_on_sc=False,
                                             needs_layout_passes=False),
    )
    final, _, _, _ = run(tab2, src2, dst2, val2)
    return final


def kernel(adj_indices, adj_values, user_table, item_table):
    table = jnp.concatenate([user_table, item_table], axis=0)
    table = jnp.pad(table, ((0, N_PAD - N), (0, 0)))
    tab2 = jnp.concatenate([table[:, :H], table[:, H:]], axis=0)

    E = adj_values.shape[0]
    e_block = NS * C * SUP
    E_pad = ((E + e_block - 1) // e_block) * e_block
    pad = E_pad - E
    dst2 = jnp.concatenate(
        [adj_indices[0], jnp.zeros((pad,), jnp.int32)]).reshape(-1, C)
    src2 = jnp.concatenate(
        [adj_indices[1], jnp.zeros((pad,), jnp.int32)]).reshape(-1, C)
    val2 = jnp.concatenate(
        [adj_values, jnp.zeros((pad,), jnp.float32)]).reshape(-1, C)

    final = _run(tab2, src2, dst2, val2, E_pad // e_block)
    all_embed = jnp.concatenate([final[:N], final[N_PAD:N_PAD + N]], axis=1)
    return (all_embed[:N_USER], all_embed[N_USER:])


# speedup vs baseline: 1.0376x; 1.0376x over previous
import functools
import jax
import jax.numpy as jnp
from jax import lax
from jax.experimental import pallas as pl
from jax.experimental.pallas import tpu as pltpu
from jax.experimental.pallas import tpu_sc as plsc

N_USER = 25000
N_ITEM = 25000
N = N_USER + N_ITEM
D = 64
H = D // 2
NS = 16
L = 16
C = 128
SUP = 16
NBUF = 5
GLEAD = 3
N_PAD = 50176
R_PER_TILE = N_PAD // NS
RC = 112


def _make_sc_body(n_supers):
  def _sc_body(tab2, src2, dst2, val2,
               final, buf0, buf1,
               acc, src_sv, dst_sv, val_sv,
               r0_v, r1_v, r2_v, r3_v, r4_v,
               g0_s, g1_s, g2_s, g3_s, g4_s,
               s0_s, s1_s, s2_s, s3_s, s4_s, i_s):
    cid = lax.axis_index("c")
    sid = lax.axis_index("s")
    rows = [r0_v, r1_v, r2_v, r3_v, r4_v]
    gsem = [g0_s, g1_s, g2_s, g3_s, g4_s]
    ssem = [s0_s, s1_s, s2_s, s3_s, s4_s]
    zeros16 = jnp.zeros((L,), jnp.float32)
    row_off = cid * N_PAD
    rbase = sid * R_PER_TILE
    erow_base = sid * (n_supers * SUP)

    def fill_zero(buf):
        def zbody(r, carry):
            buf[r, pl.ds(0, L)] = zeros16
            buf[r, pl.ds(L, L)] = zeros16
            return carry
        lax.fori_loop(0, RC, zbody, 0)

    fill_zero(r0_v)

    def clear_acc(i, carry):
        pltpu.sync_copy(r0_v.at[pl.ds(0, RC)],
                        acc.at[pl.ds(rbase + i * RC, RC)])
        return carry
    lax.fori_loop(0, R_PER_TILE // RC, clear_acc, 0)
    plsc.subcore_barrier()

    def gissue(k, r, src_tab):
        pltpu.async_copy(src_tab.at[src_sv.at[r]], rows[k], gsem[k])

    def gwait(k, src_tab):
        pltpu.make_async_copy(src_tab.at[src_sv.at[0]], rows[k],
                              gsem[k]).wait()

    def sissue(k, r):
        pltpu.async_copy(rows[k], acc.at[dst_sv.at[r]], ssem[k], add=True)

    def swait(k):
        pltpu.make_async_copy(rows[k], acc.at[dst_sv.at[0]], ssem[k]).wait()

    def scale(k, r):
        buf = rows[k]

        def gb(g, carry):
            vseg = val_sv[r, pl.ds(g * L, L)]
            for kk in range(L):
                v = vseg[kk]
                e = g * L + kk
                buf[e, pl.ds(0, L)] = buf[e, pl.ds(0, L)] * v
                buf[e, pl.ds(L, L)] = buf[e, pl.ds(L, L)] * v
            return carry
        lax.fori_loop(0, C // L, gb, 0)

    def do_edges(src_tab):
        def super_body(s, carry):
            erow0 = erow_base + s * SUP
            pltpu.async_copy(src2.at[pl.ds(erow0, SUP)], src_sv, i_s)
            pltpu.async_copy(dst2.at[pl.ds(erow0, SUP)], dst_sv, i_s)
            pltpu.async_copy(val2.at[pl.ds(erow0, SUP)], val_sv, i_s)
            pltpu.make_async_copy(src2.at[pl.ds(erow0, SUP)], src_sv,
                                  i_s).wait()
            pltpu.make_async_copy(dst2.at[pl.ds(erow0, SUP)], dst_sv,
                                  i_s).wait()
            pltpu.make_async_copy(val2.at[pl.ds(erow0, SUP)], val_sv,
                                  i_s).wait()

            def adj_body(r, c2):
                for j in range(C // L):
                    src_sv[r, pl.ds(j * L, L)] = (
                        src_sv[r, pl.ds(j * L, L)] + row_off)
                return c2
            lax.fori_loop(0, SUP, adj_body, 0)

            for k in range(GLEAD):
                gissue(k, k, src_tab)

            for t in range(SUP):
                k = t % NBUF
                ta = t + GLEAD
                if ta < SUP:
                    kb = ta % NBUF
                    if ta - NBUF >= 0:
                        swait(kb)
                    gissue(kb, ta, src_tab)
                gwait(k, src_tab)
                scale(k, t)
                sissue(k, t)
            for t in range(SUP - NBUF, SUP):
                swait(t % NBUF)
            return carry
        lax.fori_loop(0, n_supers, super_body, 0)
        plsc.subcore_barrier()

    def export_layer(dst_buf):
        fill_zero(r1_v)

        def eb(i, carry):
            r0 = rbase + i * RC
            pltpu.sync_copy(acc.at[pl.ds(r0, RC)], r0_v.at[pl.ds(0, RC)])
            pltpu.sync_copy(r0_v.at[pl.ds(0, RC)],
                            dst_buf.at[pl.ds(row_off + r0, RC)])
            pltpu.sync_copy(r1_v.at[pl.ds(0, RC)], acc.at[pl.ds(r0, RC)])
            return carry
        lax.fori_loop(0, R_PER_TILE // RC, eb, 0)
        plsc.subcore_barrier()

    do_edges(tab2)
    export_layer(buf0)
    do_edges(buf0)
    export_layer(buf1)
    do_edges(buf1)

    quarter = jnp.float32(0.25)

    def mean_body(i, carry):
        r0 = rbase + i * RC
        g0 = row_off + r0
        pltpu.sync_copy(acc.at[pl.ds(r0, RC)], r0_v.at[pl.ds(0, RC)])
        pltpu.sync_copy(tab2.at[pl.ds(g0, RC)], r1_v.at[pl.ds(0, RC)])
        pltpu.sync_copy(buf0.at[pl.ds(g0, RC)], r2_v.at[pl.ds(0, RC)])
        pltpu.sync_copy(buf1.at[pl.ds(g0, RC)], r3_v.at[pl.ds(0, RC)])

        def rbody(r, rcarry):
            for h in (0, L):
                s = (r0_v[r, pl.ds(h, L)] + r1_v[r, pl.ds(h, L)]
                     + r2_v[r, pl.ds(h, L)] + r3_v[r, pl.ds(h, L)])
                r0_v[r, pl.ds(h, L)] = s * quarter
            return rcarry
        lax.fori_loop(0, RC, rbody, 0)
        pltpu.sync_copy(r0_v.at[pl.ds(0, RC)], final.at[pl.ds(g0, RC)])
        return carry
    lax.fori_loop(0, R_PER_TILE // RC, mean_body, 0)

  return _sc_body


@functools.partial(jax.jit, static_argnames=("n_supers",))
def _run(tab2, src2, dst2, val2, n_supers):
    mesh = plsc.VectorSubcoreMesh(core_axis_name="c", subcore_axis_name="s")
    f32 = jnp.float32
    i32 = jnp.int32
    out_type = (
        jax.ShapeDtypeStruct((2 * N_PAD, H), f32),
        jax.ShapeDtypeStruct((2 * N_PAD, H), f32),
        jax.ShapeDtypeStruct((2 * N_PAD, H), f32),
    )
    scratch = (
        [pltpu.VMEM_SHARED((N_PAD, H), f32)]
        + [pltpu.VMEM((SUP, C), i32),
           pltpu.VMEM((SUP, C), i32),
           pltpu.VMEM((SUP, C), f32)]
        + [pltpu.VMEM((C, H), f32)] * NBUF
        + [pltpu.SemaphoreType.DMA] * (2 * NBUF + 1)
    )
    run = pl.kernel(
        _make_sc_body(n_supers),
        out_type=out_type,
        mesh=mesh,
        scratch_types=scratch,
        compiler_params=pltpu.CompilerParams(use_tc_tiling_on_sc=False),
    )
    final, _, _ = run(tab2, src2, dst2, val2)
    return final


def kernel(adj_indices, adj_values, user_table, item_table):
    table = jnp.concatenate([user_table, item_table], axis=0)
    table = jnp.pad(table, ((0, N_PAD - N), (0, 0)))
    tab2 = jnp.concatenate([table[:, :H], table[:, H:]], axis=0)

    E = adj_values.shape[0]
    e_block = NS * C * SUP
    E_pad = ((E + e_block - 1) // e_block) * e_block
    pad = E_pad - E
    dst2 = jnp.concatenate(
        [adj_indices[0], jnp.zeros((pad,), jnp.int32)]).reshape(-1, C)
    src2 = jnp.concatenate(
        [adj_indices[1], jnp.zeros((pad,), jnp.int32)]).reshape(-1, C)
    val2 = jnp.concatenate(
        [adj_values, jnp.zeros((pad,), jnp.float32)]).reshape(-1, C)

    final = _run(tab2, src2, dst2, val2, E_pad // e_block)
    all_embed = jnp.concatenate([final[:N], final[N_PAD:N_PAD + N]], axis=1)
    return (all_embed[:N_USER], all_embed[N_USER:])
